# baseline (speedup 1.0000x reference)
import math

import jax
import jax.numpy as jnp
from jax.experimental import pallas as pl
from jax.experimental.pallas import tpu as pltpu


def _round_up(x: int, m: int) -> int:
    return ((x + m - 1) // m) * m


def _gram_body(x1_ref, x2_ref, out_ref):
    a = x1_ref[...].astype(jnp.bfloat16)
    b = x2_ref[...].astype(jnp.bfloat16)
    out_ref[...] = jax.lax.dot_general(
        a,
        b,
        dimension_numbers=(((2,), (2,)), ((0,), (0,))),
        preferred_element_type=jnp.float32,
    )


def kernel(X1: jax.Array, X2: jax.Array) -> jax.Array:
    N, D = X1.shape[-2], X1.shape[-1]
    M = X2.shape[-2]
    batch_shape = jnp.broadcast_shapes(X1.shape[:-2], X2.shape[:-2])
    B = math.prod(batch_shape) if batch_shape else 1

    x1 = jnp.broadcast_to(X1.astype(jnp.float32), (*batch_shape, N, D)).reshape(B, N, D)
    x2 = jnp.broadcast_to(X2.astype(jnp.float32), (*batch_shape, M, D)).reshape(B, M, D)

    assert B % 2 == 0 and N % 512 == 0
    bt = B // 2
    tn = 512

    out = pl.pallas_call(
        _gram_body,
        out_shape=jax.ShapeDtypeStruct((B, N, M), jnp.float32),
        grid=(2, N // tn),
        in_specs=[
            pl.BlockSpec((bt, tn, D), lambda i, n: (i, n, 0)),
            pl.BlockSpec((bt, M, D), lambda i, n: (i, 0, 0)),
        ],
        out_specs=pl.BlockSpec((bt, tn, M), lambda i, n: (i, n, 0)),
        compiler_params=pltpu.CompilerParams(
            dimension_semantics=("parallel", "arbitrary"),
            vmem_limit_bytes=60 * 1024 * 1024,
        ),
        cost_estimate=pl.CostEstimate(
            flops=2 * B * N * M * D,
            transcendentals=0,
            bytes_accessed=4 * B * ((N + M) * D + N * M),
        ),
    )(x1, x2)

    return out.reshape(*batch_shape, N, M)
